# baseline (speedup 1.0000x reference)
import functools

import jax
import jax.numpy as jnp
from jax import lax
from jax.experimental import pallas as pl
from jax.experimental.pallas import tpu as pltpu
from jax.experimental.pallas import tpu_sc as plsc

_B = 4096
_L = 20
_H = 64
_V = 1000

_NC = 2
_NS = 16
_NW = _NC * _NS
_BPW = _B // _NW
_G = 16
_NG = _BPW // _G


def _sc_pool_body(ids_hbm, table_hbm, out_hbm, table_v, ids_v, acc_v, out_v):
    wid = lax.axis_index("c") * _NS + lax.axis_index("s")
    base_b = wid * _BPW
    pltpu.sync_copy(table_hbm, table_v)
    pltpu.sync_copy(ids_hbm.at[pl.ds(base_b * _L, _BPW * _L)], ids_v)

    lane = lax.broadcasted_iota(jnp.int32, (_G,), 0)
    lane_l = lane * _L
    lane_h = lane * _H

    def group_body(g, carry):
        goff = g * (_G * _L)

        def ids_at(l):
            return plsc.load_gather(ids_v, [goff + lane_l + l])

        base0 = ids_at(0) * _H
        for c in range(_H):
            acc_v[c] = plsc.load_gather(table_v, [base0 + c])

        def l_body(l, c2):
            basev = ids_at(l) * _H
            for c in range(_H):
                plsc.addupdate(acc_v.at[c], plsc.load_gather(table_v, [basev + c]))
            return c2

        lax.fori_loop(1, _L, l_body, 0)

        zero = lane * 0
        for c in range(_H):
            plsc.store_scatter(out_v, [lane, zero + c], acc_v[c])
        pltpu.sync_copy(out_v, out_hbm.at[pl.ds(base_b + g * _G, _G), :])
        return carry

    lax.fori_loop(0, _NG, group_body, 0)


@functools.partial(
    pl.kernel,
    out_type=jax.ShapeDtypeStruct((_B, _H), jnp.float32),
    mesh=plsc.VectorSubcoreMesh(core_axis_name="c", subcore_axis_name="s"),
    compiler_params=pltpu.CompilerParams(needs_layout_passes=False),
    scratch_types=[
        pltpu.VMEM((_V * _H,), jnp.float32),
        pltpu.VMEM((_BPW * _L,), jnp.int32),
        pltpu.VMEM((_H, _G), jnp.float32),
        pltpu.VMEM((_G, _H), jnp.float32),
    ],
)
def _sc_pool(ids_hbm, table_hbm, out_hbm, table_v, ids_v, acc_v, out_v):
    _sc_pool_body(ids_hbm, table_hbm, out_hbm, table_v, ids_v, acc_v, out_v)


_TB = 256


def _tc_body(x_ref, w_ref, b_ref, y_ref):
    x = x_ref[:, :] * (1.0 / _L)
    y = jnp.dot(x, w_ref[:, :], preferred_element_type=jnp.float32)
    y_ref[:, :] = y + b_ref[:, :]


def _tc_project(pooled, W, b2d):
    return pl.pallas_call(
        _tc_body,
        grid=(_B // _TB,),
        in_specs=[
            pl.BlockSpec((_TB, _H), lambda i: (i, 0)),
            pl.BlockSpec((_H, _V), lambda i: (0, 0)),
            pl.BlockSpec((1, _V), lambda i: (0, 0)),
        ],
        out_specs=pl.BlockSpec((_TB, _V), lambda i: (i, 0)),
        out_shape=jax.ShapeDtypeStruct((_B, _V), jnp.float32),
        compiler_params=pltpu.CompilerParams(
            dimension_semantics=("parallel",)),
    )(pooled, W, b2d)


_NSLOT = 4


def _sc_bcast_body(y_hbm, out_hbm, idx_v, stg_v, *sems):
    gsems, wsems = sems[:_NSLOT], sems[_NSLOT:]
    wid = lax.axis_index("c") * _NS + lax.axis_index("s")
    base_b = wid * _BPW
    lane16 = lax.broadcasted_iota(jnp.int32, (16,), 0)

    def build_idx(k, c):
        bval = lane16 * 0 + (base_b + k)
        idx_v[k, pl.ds(0, 16)] = bval
        plsc.store_scatter(idx_v.at[k], [lane16 + 4], bval)
        return c

    lax.fori_loop(0, _BPW, build_idx, 0)

    def gather_k(k, slot, sem):
        return pltpu.make_async_copy(
            y_hbm.at[idx_v.at[k]], stg_v.at[slot], sem)

    def write_k(k, slot, sem):
        return pltpu.make_async_copy(
            stg_v.at[slot], out_hbm.at[base_b + k], sem)

    def quad(q, carry):
        for j in range(_NSLOT):
            k = q * _NSLOT + j
            pj = (j - 1) % _NSLOT
            @pl.when(k >= _NSLOT)
            def _():
                write_k(0, j, wsems[j]).wait()
            gather_k(k, j, gsems[j]).start()
            @pl.when(k >= 1)
            def _():
                gather_k(0, pj, gsems[pj]).wait()
                write_k(k - 1, pj, wsems[pj]).start()
        return carry

    lax.fori_loop(0, _BPW // _NSLOT, quad, 0)
    last = _NSLOT - 1
    gather_k(0, last, gsems[last]).wait()
    write_k(_BPW - 1, last, wsems[last]).start()
    for j in range(_NSLOT):
        write_k(0, j, wsems[j]).wait()


@functools.partial(
    pl.kernel,
    out_type=jax.ShapeDtypeStruct((_B, _L, _V), jnp.float32),
    mesh=plsc.VectorSubcoreMesh(core_axis_name="c", subcore_axis_name="s"),
    compiler_params=pltpu.CompilerParams(
        needs_layout_passes=False, use_tc_tiling_on_sc=False),
    scratch_types=[
        pltpu.VMEM((_BPW, _L), jnp.int32),
        pltpu.VMEM((_NSLOT, _L, _V), jnp.float32),
    ] + [pltpu.SemaphoreType.DMA] * (2 * _NSLOT),
)
def _sc_bcast(y_hbm, out_hbm, idx_v, stg_v, *sems):
    _sc_bcast_body(y_hbm, out_hbm, idx_v, stg_v, *sems)


def kernel(input_ids, emb_table, W, b):
    ids_flat = input_ids.astype(jnp.int32).reshape(-1)
    table_flat = emb_table.reshape(-1)
    pooled = _sc_pool(ids_flat, table_flat)
    y = _tc_project(pooled, W, b.reshape(1, _V))
    return _sc_bcast(y)


# speedup vs baseline: 1.1186x; 1.1186x over previous
import functools

import jax
import jax.numpy as jnp
from jax import lax
from jax.experimental import pallas as pl
from jax.experimental.pallas import tpu as pltpu
from jax.experimental.pallas import tpu_sc as plsc

_B = 4096
_L = 20
_H = 64
_V = 1000

_NC = 2
_NS = 16
_NW = _NC * _NS
_BPW = _B // _NW
_G = 16
_NG = _BPW // _G


def _sc_pool_body(ids_hbm, table_hbm, out_hbm, table_v, ids_v, acc_v, out_v):
    wid = lax.axis_index("c") * _NS + lax.axis_index("s")
    base_b = wid * _BPW
    pltpu.sync_copy(table_hbm, table_v)
    pltpu.sync_copy(ids_hbm.at[pl.ds(base_b * _L, _BPW * _L)], ids_v)

    lane = lax.broadcasted_iota(jnp.int32, (_G,), 0)
    lane_l = lane * _L
    lane_h = lane * _H

    def group_body(g, carry):
        goff = g * (_G * _L)

        def ids_at(l):
            return plsc.load_gather(ids_v, [goff + lane_l + l])

        base0 = ids_at(0) * _H
        for c in range(_H):
            acc_v[c] = plsc.load_gather(table_v, [base0 + c])

        def l_body(l, c2):
            basev = ids_at(l) * _H
            for c in range(_H):
                plsc.addupdate(acc_v.at[c], plsc.load_gather(table_v, [basev + c]))
            return c2

        lax.fori_loop(1, _L, l_body, 0)

        zero = lane * 0
        for c in range(_H):
            plsc.store_scatter(out_v, [lane, zero + c], acc_v[c])
        pltpu.sync_copy(out_v, out_hbm.at[pl.ds(base_b + g * _G, _G), :])
        return carry

    lax.fori_loop(0, _NG, group_body, 0)


@functools.partial(
    pl.kernel,
    out_type=jax.ShapeDtypeStruct((_B, _H), jnp.float32),
    mesh=plsc.VectorSubcoreMesh(core_axis_name="c", subcore_axis_name="s"),
    compiler_params=pltpu.CompilerParams(needs_layout_passes=False),
    scratch_types=[
        pltpu.VMEM((_V * _H,), jnp.float32),
        pltpu.VMEM((_BPW * _L,), jnp.int32),
        pltpu.VMEM((_H, _G), jnp.float32),
        pltpu.VMEM((_G, _H), jnp.float32),
    ],
)
def _sc_pool(ids_hbm, table_hbm, out_hbm, table_v, ids_v, acc_v, out_v):
    _sc_pool_body(ids_hbm, table_hbm, out_hbm, table_v, ids_v, acc_v, out_v)


_TB = 256


def _tc_body(x_ref, w_ref, b_ref, y_ref):
    x = x_ref[:, :] * (1.0 / _L)
    y = jnp.dot(x, w_ref[:, :], preferred_element_type=jnp.float32)
    y_ref[:, :] = y + b_ref[:, :]


def _tc_project(pooled, W, b2d):
    return pl.pallas_call(
        _tc_body,
        grid=(_B // _TB,),
        in_specs=[
            pl.BlockSpec((_TB, _H), lambda i: (i, 0)),
            pl.BlockSpec((_H, _V), lambda i: (0, 0)),
            pl.BlockSpec((1, _V), lambda i: (0, 0)),
        ],
        out_specs=pl.BlockSpec((_TB, _V), lambda i: (i, 0)),
        out_shape=jax.ShapeDtypeStruct((_B, _V), jnp.float32),
        compiler_params=pltpu.CompilerParams(
            dimension_semantics=("parallel",)),
    )(pooled, W, b2d)


_NSLOT = 2
_YBLK = 16
_CHUNKS = tuple(16 * j for j in range(_V // 16)) + (_V - 16,)


def _sc_bcast_body(y_hbm, out_hbm, y_v, stg_v, *sems):
    wid = lax.axis_index("c") * _NS + lax.axis_index("s")
    base_b = wid * _BPW

    def write_k(k, slot, sem):
        return pltpu.make_async_copy(
            stg_v.at[slot], out_hbm.at[base_b + k], sem)

    def do_row(k, rl, slot):
        @pl.when(k >= _NSLOT)
        def _():
            write_k(0, slot, sems[slot]).wait()

        def l_body(l, c):
            for off in _CHUNKS:
                stg_v[slot, l, pl.ds(off, 16)] = y_v[rl, pl.ds(off, 16)]
            return c

        lax.fori_loop(0, _L, l_body, 0)
        write_k(k, slot, sems[slot]).start()

    def oblk(o, carry):
        pltpu.sync_copy(y_hbm.at[pl.ds(base_b + o * _YBLK, _YBLK), :], y_v)

        def pair(p, c2):
            for slot in range(_NSLOT):
                rl = p * _NSLOT + slot
                do_row(o * _YBLK + rl, rl, slot)
            return c2

        lax.fori_loop(0, _YBLK // _NSLOT, pair, 0)
        return carry

    lax.fori_loop(0, _BPW // _YBLK, oblk, 0)
    for slot in range(_NSLOT):
        write_k(0, slot, sems[slot]).wait()


@functools.partial(
    pl.kernel,
    out_type=jax.ShapeDtypeStruct((_B, _L, _V), jnp.float32),
    mesh=plsc.VectorSubcoreMesh(core_axis_name="c", subcore_axis_name="s"),
    compiler_params=pltpu.CompilerParams(needs_layout_passes=False),
    scratch_types=[
        pltpu.VMEM((_YBLK, _V), jnp.float32),
        pltpu.VMEM((_NSLOT, _L, _V), jnp.float32),
    ] + [pltpu.SemaphoreType.DMA] * _NSLOT,
)
def _sc_bcast(y_hbm, out_hbm, y_v, stg_v, *sems):
    _sc_bcast_body(y_hbm, out_hbm, y_v, stg_v, *sems)


def kernel(input_ids, emb_table, W, b):
    ids_flat = input_ids.astype(jnp.int32).reshape(-1)
    table_flat = emb_table.reshape(-1)
    pooled = _sc_pool(ids_flat, table_flat)
    y = _tc_project(pooled, W, b.reshape(1, _V))
    return _sc_bcast(y)


# speedup vs baseline: 1.9706x; 1.7617x over previous
import functools

import jax
import jax.numpy as jnp
from jax import lax
from jax.experimental import pallas as pl
from jax.experimental.pallas import tpu as pltpu
from jax.experimental.pallas import tpu_sc as plsc

_B = 4096
_L = 20
_H = 64
_V = 1000

_NC = 2
_NS = 16
_NW = _NC * _NS
_BPW = _B // _NW
_G = 16
_NG = _BPW // _G


def _sc_pool_body(ids_hbm, table_hbm, out_hbm, table_v, ids_v, acc_v, out_v):
    wid = lax.axis_index("c") * _NS + lax.axis_index("s")
    base_b = wid * _BPW
    pltpu.sync_copy(table_hbm, table_v)
    pltpu.sync_copy(ids_hbm.at[pl.ds(base_b * _L, _BPW * _L)], ids_v)

    lane = lax.broadcasted_iota(jnp.int32, (_G,), 0)
    lane_l = lane * _L
    lane_h = lane * _H

    def group_body(g, carry):
        goff = g * (_G * _L)

        def ids_at(l):
            return plsc.load_gather(ids_v, [goff + lane_l + l])

        base0 = ids_at(0) * _H
        for c in range(_H):
            acc_v[c] = plsc.load_gather(table_v, [base0 + c])

        def l_body(l, c2):
            basev = ids_at(l) * _H
            for c in range(_H):
                plsc.addupdate(acc_v.at[c], plsc.load_gather(table_v, [basev + c]))
            return c2

        lax.fori_loop(1, _L, l_body, 0)

        zero = lane * 0
        for c in range(_H):
            plsc.store_scatter(out_v, [lane, zero + c], acc_v[c])
        pltpu.sync_copy(out_v, out_hbm.at[pl.ds(base_b + g * _G, _G), :])
        return carry

    lax.fori_loop(0, _NG, group_body, 0)


@functools.partial(
    pl.kernel,
    out_type=jax.ShapeDtypeStruct((_B, _H), jnp.float32),
    mesh=plsc.VectorSubcoreMesh(core_axis_name="c", subcore_axis_name="s"),
    compiler_params=pltpu.CompilerParams(needs_layout_passes=False),
    scratch_types=[
        pltpu.VMEM((_V * _H,), jnp.float32),
        pltpu.VMEM((_BPW * _L,), jnp.int32),
        pltpu.VMEM((_H, _G), jnp.float32),
        pltpu.VMEM((_G, _H), jnp.float32),
    ],
)
def _sc_pool(ids_hbm, table_hbm, out_hbm, table_v, ids_v, acc_v, out_v):
    _sc_pool_body(ids_hbm, table_hbm, out_hbm, table_v, ids_v, acc_v, out_v)


_TB = 64
_NQ = 4


def _tc_body(x_ref, w_ref, b_ref, out_ref, bc_ref, sems):
    i = pl.program_id(0)
    nsteps = pl.num_programs(0)
    slot = lax.rem(i, _NQ)

    @pl.when(i >= _NQ)
    def _():
        pltpu.make_async_copy(
            bc_ref.at[slot], out_ref.at[pl.ds(0, _TB)], sems.at[slot]
        ).wait()

    x = x_ref[:, :] * (1.0 / _L)
    y = jnp.dot(x, w_ref[:, :], preferred_element_type=jnp.float32)
    y = y + b_ref[:, :]
    for l in range(_L):
        bc_ref[slot, :, l, :] = y
    pltpu.make_async_copy(
        bc_ref.at[slot], out_ref.at[pl.ds(i * _TB, _TB)], sems.at[slot]
    ).start()

    @pl.when(i == nsteps - 1)
    def _():
        for q in range(_NQ):
            pltpu.make_async_copy(
                bc_ref.at[q], out_ref.at[pl.ds(0, _TB)], sems.at[q]
            ).wait()


def _tc_project(pooled, W, b2d):
    return pl.pallas_call(
        _tc_body,
        grid=(_B // _TB,),
        in_specs=[
            pl.BlockSpec((_TB, _H), lambda i: (i, 0)),
            pl.BlockSpec((_H, _V), lambda i: (0, 0)),
            pl.BlockSpec((1, _V), lambda i: (0, 0)),
        ],
        out_specs=pl.BlockSpec(memory_space=pl.ANY),
        out_shape=jax.ShapeDtypeStruct((_B, _L, _V), jnp.float32),
        scratch_shapes=[
            pltpu.VMEM((_NQ, _TB, _L, _V), jnp.float32),
            pltpu.SemaphoreType.DMA((_NQ,)),
        ],
        compiler_params=pltpu.CompilerParams(
            dimension_semantics=("arbitrary",)),
    )(pooled, W, b2d)


def kernel(input_ids, emb_table, W, b):
    ids_flat = input_ids.astype(jnp.int32).reshape(-1)
    table_flat = emb_table.reshape(-1)
    pooled = _sc_pool(ids_flat, table_flat)
    return _tc_project(pooled, W, b.reshape(1, _V))


# speedup vs baseline: 2.4331x; 1.2347x over previous
import functools

import jax
import jax.numpy as jnp
from jax import lax
from jax.experimental import pallas as pl
from jax.experimental.pallas import tpu as pltpu
from jax.experimental.pallas import tpu_sc as plsc

_B = 4096
_L = 20
_H = 64
_V = 1000

_NC = 2
_NS = 16
_NW = _NC * _NS
_BPW = _B // _NW
_G = 16
_NG = _BPW // _G


def _sc_pool_body(ids_hbm, table_hbm, out_hbm, table_v, ids_v, out_v):
    wid = lax.axis_index("c") * _NS + lax.axis_index("s")
    base_b = wid * _BPW
    pltpu.sync_copy(table_hbm, table_v)
    pltpu.sync_copy(ids_hbm.at[pl.ds(base_b, _BPW), :], ids_v)

    def row_body(r, carry):
        v0 = ids_v[r, pl.ds(0, 16)] * _H
        v1 = ids_v[r, pl.ds(16, 16)] * _H
        offs = [v0[i] for i in range(16)] + [v1[i] for i in range(_L - 16)]
        for g in range(_H // 16):
            acc = table_v[pl.ds(offs[0] + g * 16, 16)]
            for l in range(1, _L):
                acc = acc + table_v[pl.ds(offs[l] + g * 16, 16)]
            out_v[r, pl.ds(g * 16, 16)] = acc
        return carry

    lax.fori_loop(0, _BPW, row_body, 0)
    pltpu.sync_copy(out_v, out_hbm.at[pl.ds(base_b, _BPW), :])


@functools.partial(
    pl.kernel,
    out_type=jax.ShapeDtypeStruct((_B, _H), jnp.float32),
    mesh=plsc.VectorSubcoreMesh(core_axis_name="c", subcore_axis_name="s"),
    compiler_params=pltpu.CompilerParams(needs_layout_passes=False),
    scratch_types=[
        pltpu.VMEM((_V * _H,), jnp.float32),
        pltpu.VMEM((_BPW, 32), jnp.int32),
        pltpu.VMEM((_BPW, _H), jnp.float32),
    ],
)
def _sc_pool(ids_hbm, table_hbm, out_hbm, table_v, ids_v, out_v):
    _sc_pool_body(ids_hbm, table_hbm, out_hbm, table_v, ids_v, out_v)


_TB = 64
_NQ = 4


def _tc_body(x_ref, w_ref, b_ref, out_ref, bc_ref, sems):
    i = pl.program_id(0)
    nsteps = pl.num_programs(0)
    slot = lax.rem(i, _NQ)

    @pl.when(i >= _NQ)
    def _():
        pltpu.make_async_copy(
            bc_ref.at[slot], out_ref.at[pl.ds(0, _TB)], sems.at[slot]
        ).wait()

    x = x_ref[:, :] * (1.0 / _L)
    y = jnp.dot(x, w_ref[:, :], preferred_element_type=jnp.float32)
    y = y + b_ref[:, :]
    for l in range(_L):
        bc_ref[slot, :, l, :] = y
    pltpu.make_async_copy(
        bc_ref.at[slot], out_ref.at[pl.ds(i * _TB, _TB)], sems.at[slot]
    ).start()

    @pl.when(i == nsteps - 1)
    def _():
        for q in range(_NQ):
            pltpu.make_async_copy(
                bc_ref.at[q], out_ref.at[pl.ds(0, _TB)], sems.at[q]
            ).wait()


def _tc_project(pooled, W, b2d):
    return pl.pallas_call(
        _tc_body,
        grid=(_B // _TB,),
        in_specs=[
            pl.BlockSpec((_TB, _H), lambda i: (i, 0)),
            pl.BlockSpec((_H, _V), lambda i: (0, 0)),
            pl.BlockSpec((1, _V), lambda i: (0, 0)),
        ],
        out_specs=pl.BlockSpec(memory_space=pl.ANY),
        out_shape=jax.ShapeDtypeStruct((_B, _L, _V), jnp.float32),
        scratch_shapes=[
            pltpu.VMEM((_NQ, _TB, _L, _V), jnp.float32),
            pltpu.SemaphoreType.DMA((_NQ,)),
        ],
        compiler_params=pltpu.CompilerParams(
            dimension_semantics=("arbitrary",)),
    )(pooled, W, b2d)


def kernel(input_ids, emb_table, W, b):
    ids_pad = jnp.pad(input_ids.astype(jnp.int32), ((0, 0), (0, 32 - _L)))
    table_flat = emb_table.reshape(-1)
    pooled = _sc_pool(ids_pad, table_flat)
    return _tc_project(pooled, W, b.reshape(1, _V))
